# baseline (speedup 1.0000x reference)
import jax
import jax.numpy as jnp
from jax.experimental import pallas as pl
from jax.experimental.pallas import tpu as pltpu

GSTEPS = 4


def _make_body(bp, n1, n2):
    def body(x1_ref, x2_ref, out_ref):
        step = pl.program_id(0)
        for i in range(bp):
            s1 = x1_ref[i * n1:(i + 1) * n1, :]
            s2 = x2_ref[i * n2:(i + 1) * n2, :]
            g = jax.lax.dot_general(
                s1, s2, (((1,), (1,)), ((), ())),
                preferred_element_type=jnp.float32,
                precision=jax.lax.Precision.DEFAULT,
            )
            q1 = jnp.sum(s1 * s1, axis=1)
            q2 = jnp.sum(s2 * s2, axis=1)
            dist = q1[:, None] + q2[None, :] - 2.0 * g
            a = jnp.sum(jnp.min(dist, axis=0))
            b = jnp.sum(jnp.min(dist, axis=1))
            out_ref[step * bp + i] = (a + b) / jnp.float32(min(n1, n2))
    return body


def kernel(x1, edge_index1, sz1, x2, edge_index2, sz2):
    del edge_index1, edge_index2
    B = sz1.shape[0]
    N1, D = x1.shape
    N2 = x2.shape[0]
    n1 = N1 // B
    n2 = N2 // B
    del sz2
    bp = B // GSTEPS
    out = pl.pallas_call(
        _make_body(bp, n1, n2),
        grid=(GSTEPS,),
        in_specs=[
            pl.BlockSpec((N1 // GSTEPS, D), lambda i: (i, 0)),
            pl.BlockSpec((N2 // GSTEPS, D), lambda i: (i, 0)),
        ],
        out_specs=pl.BlockSpec(memory_space=pltpu.SMEM),
        out_shape=jax.ShapeDtypeStruct((B,), jnp.float32),
    )(x1, x2)
    return out


# speedup vs baseline: 1.2412x; 1.2412x over previous
import jax
import jax.numpy as jnp
from jax.experimental import pallas as pl
from jax.experimental.pallas import tpu as pltpu

GSTEPS = 2


def _make_body(bp, n1, n2):
    def body(x1_ref, x2_ref, out_ref):
        step = pl.program_id(0)
        dims = (((1,), (1,)), ((), ()))
        d_feat = x1_ref.shape[1]
        ones_r = jnp.ones((n2, d_feat), jnp.float32)
        ones_l = jnp.ones((n1, d_feat), jnp.float32)
        for i in range(bp):
            s1 = x1_ref[i * n1:(i + 1) * n1, :]
            s2 = x2_ref[i * n2:(i + 1) * n2, :]
            g = jax.lax.dot_general(
                s1, s2, dims,
                preferred_element_type=jnp.float32,
                precision=jax.lax.Precision.DEFAULT,
            )
            q1m = jax.lax.dot_general(
                s1 * s1, ones_r, dims,
                preferred_element_type=jnp.float32,
                precision=jax.lax.Precision.DEFAULT,
            )
            q2m = jax.lax.dot_general(
                ones_l, s2 * s2, dims,
                preferred_element_type=jnp.float32,
                precision=jax.lax.Precision.DEFAULT,
            )
            dist = (q1m + q2m) - 2.0 * g
            a = jnp.sum(jnp.min(dist, axis=0))
            b = jnp.sum(jnp.min(dist, axis=1))
            out_ref[step * bp + i] = (a + b) / jnp.float32(min(n1, n2))
    return body


def kernel(x1, edge_index1, sz1, x2, edge_index2, sz2):
    del edge_index1, edge_index2
    B = sz1.shape[0]
    N1, D = x1.shape
    N2 = x2.shape[0]
    n1 = N1 // B
    n2 = N2 // B
    del sz2
    bp = B // GSTEPS
    out = pl.pallas_call(
        _make_body(bp, n1, n2),
        grid=(GSTEPS,),
        in_specs=[
            pl.BlockSpec((N1 // GSTEPS, D), lambda i: (i, 0)),
            pl.BlockSpec((N2 // GSTEPS, D), lambda i: (i, 0)),
        ],
        out_specs=pl.BlockSpec(memory_space=pltpu.SMEM),
        out_shape=jax.ShapeDtypeStruct((B,), jnp.float32),
    )(x1, x2)
    return out
